# baseline (speedup 1.0000x reference)
import functools

import jax
import jax.numpy as jnp
from jax import lax
from jax.experimental import pallas as pl
from jax.experimental.pallas import tpu as pltpu
from jax.experimental.pallas import tpu_sc as plsc

_B = 16
_P = 4096
_S = 1024
_L = 16
_CH = _P // _L


def _fps_body(x_hbm, y_hbm, z_hbm, out_hbm, x_v, y_v, z_v, dist_v, idx_v):
    c = lax.axis_index("c")
    s = lax.axis_index("s")
    b = s

    @pl.when(c == 0)
    def _():
        lanes = lax.iota(jnp.int32, _L)
        pltpu.sync_copy(x_hbm.at[pl.ds(b * _P, _P)], x_v)
        pltpu.sync_copy(y_hbm.at[pl.ds(b * _P, _P)], y_v)
        pltpu.sync_copy(z_hbm.at[pl.ds(b * _P, _P)], z_v)

        def init_chunk(i, _):
            for u in range(8):
                dist_v[pl.ds(i * (8 * _L) + u * _L, _L)] = jnp.full(
                    (_L,), jnp.inf, jnp.float32)
            return 0

        lax.fori_loop(jnp.int32(0), jnp.int32(_CH // 8), init_chunk, 0)

        plsc.store_scatter(idx_v, [jnp.zeros((_L,), jnp.int32)],
                           jnp.full((_L,), b * _P, jnp.int32),
                           mask=lanes == 0)

        def outer(i, sel):
            selv = jnp.full((_L,), sel, jnp.int32)
            sx = plsc.load_gather(x_v, [selv])
            sy = plsc.load_gather(y_v, [selv])
            sz = plsc.load_gather(z_v, [selv])

            def chunk(k8, carry):
                rmaxs, ridxs = carry
                new_rmaxs, new_ridxs = [], []
                for u in range(8):
                    k = k8 * 8 + u
                    sl = pl.ds(k * _L, _L)
                    dx = x_v[sl] - sx
                    dy = y_v[sl] - sy
                    dz = z_v[sl] - sz
                    d = dx * dx + dy * dy + dz * dz
                    dmin = jnp.minimum(dist_v[sl], d)
                    dist_v[sl] = dmin
                    pred = dmin > rmaxs[u]
                    new_rmaxs.append(jnp.where(pred, dmin, rmaxs[u]))
                    new_ridxs.append(
                        jnp.where(pred, jnp.full((_L,), k, jnp.int32),
                                  ridxs[u]))
                return tuple(new_rmaxs), tuple(new_ridxs)

            neg = jnp.full((_L,), -1.0, jnp.float32)
            zero = jnp.zeros((_L,), jnp.int32)
            rmaxs, ridxs = lax.fori_loop(
                jnp.int32(0), jnp.int32(_CH // 8), chunk,
                ((neg,) * 8, (zero,) * 8))

            m8 = rmaxs[0]
            for u in range(1, 8):
                m8 = jnp.maximum(m8, rmaxs[u])
            m = jnp.max(m8)
            big = jnp.full((_L,), 2 ** 30, jnp.int32)
            cand = big
            for u in range(8):
                lin = ridxs[u] * _L + lanes
                cand = jnp.minimum(cand, jnp.where(rmaxs[u] == m, lin, big))
            nsel = jnp.min(cand)
            plsc.store_scatter(idx_v, [jnp.full((_L,), i, jnp.int32)],
                               jnp.full((_L,), b * _P + nsel, jnp.int32),
                               mask=lanes == 0)
            return nsel

        lax.fori_loop(jnp.int32(1), jnp.int32(_S), outer, jnp.int32(0))
        pltpu.sync_copy(idx_v, out_hbm.at[b])


_fps_kernel = functools.partial(
    pl.kernel,
    out_type=jax.ShapeDtypeStruct((_B, _S), jnp.int32),
    mesh=plsc.VectorSubcoreMesh(core_axis_name="c", subcore_axis_name="s",
                                num_cores=2, num_subcores=16),
    compiler_params=pltpu.CompilerParams(needs_layout_passes=False),
    scratch_types=[
        pltpu.VMEM((_P,), jnp.float32),
        pltpu.VMEM((_P,), jnp.float32),
        pltpu.VMEM((_P,), jnp.float32),
        pltpu.VMEM((_P,), jnp.float32),
        pltpu.VMEM((_S,), jnp.int32),
    ],
)(_fps_body)


def kernel(pos, batch):
    del batch
    x = pos[:, 0]
    y = pos[:, 1]
    z = pos[:, 2]
    idx = _fps_kernel(x, y, z)
    return idx.reshape(-1).astype(jnp.int64)


# speedup vs baseline: 1.0001x; 1.0001x over previous
import functools

import jax
import jax.numpy as jnp
from jax import lax
from jax.experimental import pallas as pl
from jax.experimental.pallas import tpu as pltpu
from jax.experimental.pallas import tpu_sc as plsc

_B = 16
_P = 4096
_S = 1024
_L = 16
_CH = _P // _L
_U = 8


def _fps_body(x_hbm, y_hbm, z_hbm, out_hbm,
              x_v, y_v, z_v, da_v, db_v, idx_v):
    c = lax.axis_index("c")
    s = lax.axis_index("s")
    b = s

    @pl.when(c == 0)
    def _():
        lanes = lax.iota(jnp.int32, _L)
        pltpu.sync_copy(x_hbm.at[pl.ds(b * _P, _P)], x_v)
        pltpu.sync_copy(y_hbm.at[pl.ds(b * _P, _P)], y_v)
        pltpu.sync_copy(z_hbm.at[pl.ds(b * _P, _P)], z_v)

        plsc.store_scatter(idx_v, [jnp.zeros((_L,), jnp.int32)],
                           jnp.full((_L,), b * _P, jnp.int32),
                           mask=lanes == 0)

        def fused_pass(src, dst, sel, i):
            selv = jnp.full((_L,), sel, jnp.int32)
            sx = plsc.load_gather(x_v, [selv])
            sy = plsc.load_gather(y_v, [selv])
            sz = plsc.load_gather(z_v, [selv])

            def chunk(k8, carry):
                rmaxs, ridxs = carry
                new_rmaxs, new_ridxs = [], []
                for u in range(_U):
                    k = k8 * _U + u
                    sl = pl.ds(k * _L, _L)
                    dx = x_v[sl] - sx
                    dy = y_v[sl] - sy
                    dz = z_v[sl] - sz
                    d = dx * dx + dy * dy + dz * dz
                    dmin = d if src is None else jnp.minimum(src[sl], d)
                    dst[sl] = dmin
                    pred = dmin > rmaxs[u]
                    new_rmaxs.append(jnp.where(pred, dmin, rmaxs[u]))
                    new_ridxs.append(
                        jnp.where(pred, jnp.full((_L,), k, jnp.int32),
                                  ridxs[u]))
                return tuple(new_rmaxs), tuple(new_ridxs)

            neg = jnp.full((_L,), -1.0, jnp.float32)
            zero = jnp.zeros((_L,), jnp.int32)
            rmaxs, ridxs = lax.fori_loop(
                jnp.int32(0), jnp.int32(_CH // _U), chunk,
                ((neg,) * _U, (zero,) * _U))

            m8 = rmaxs[0]
            for u in range(1, _U):
                m8 = jnp.maximum(m8, rmaxs[u])
            m = jnp.max(m8)
            big = jnp.full((_L,), 2 ** 30, jnp.int32)
            cand = big
            for u in range(_U):
                lin = ridxs[u] * _L + lanes
                cand = jnp.minimum(cand, jnp.where(rmaxs[u] == m, lin, big))
            nsel = jnp.min(cand)
            plsc.store_scatter(idx_v, [jnp.full((_L,), i, jnp.int32)],
                               jnp.full((_L,), b * _P + nsel, jnp.int32),
                               mask=lanes == 0)
            return nsel

        sel = fused_pass(None, da_v, jnp.int32(0), jnp.int32(1))

        def pair(j, sel):
            sel = fused_pass(da_v, db_v, sel, 2 * j + 2)
            sel = fused_pass(db_v, da_v, sel, 2 * j + 3)
            return sel

        lax.fori_loop(jnp.int32(0), jnp.int32((_S - 2) // 2), pair, sel)
        pltpu.sync_copy(idx_v, out_hbm.at[b])


_fps_kernel = functools.partial(
    pl.kernel,
    out_type=jax.ShapeDtypeStruct((_B, _S), jnp.int32),
    mesh=plsc.VectorSubcoreMesh(core_axis_name="c", subcore_axis_name="s",
                                num_cores=2, num_subcores=16),
    compiler_params=pltpu.CompilerParams(needs_layout_passes=False),
    scratch_types=[
        pltpu.VMEM((_P,), jnp.float32),
        pltpu.VMEM((_P,), jnp.float32),
        pltpu.VMEM((_P,), jnp.float32),
        pltpu.VMEM((_P,), jnp.float32),
        pltpu.VMEM((_P,), jnp.float32),
        pltpu.VMEM((_S,), jnp.int32),
    ],
)(_fps_body)


def kernel(pos, batch):
    del batch
    x = pos[:, 0]
    y = pos[:, 1]
    z = pos[:, 2]
    idx = _fps_kernel(x, y, z)
    return idx.reshape(-1).astype(jnp.int64)


# speedup vs baseline: 3.3646x; 3.3643x over previous
import functools

import jax
import jax.numpy as jnp
from jax import lax
from jax.experimental import pallas as pl
from jax.experimental.pallas import tpu as pltpu
from jax.experimental.pallas import tpu_sc as plsc

_B = 16
_P = 4096
_S = 1024
_L = 16
_CH = _P // _L


def _fps_body(x_hbm, y_hbm, z_hbm, out_hbm, x_v, y_v, z_v, dist_v, idx_v):
    c = lax.axis_index("c")
    s = lax.axis_index("s")
    b = s

    @pl.when(c == 0)
    def _():
        lanes = lax.iota(jnp.int32, _L)
        pltpu.sync_copy(x_hbm.at[pl.ds(b * _P, _P)], x_v)
        pltpu.sync_copy(y_hbm.at[pl.ds(b * _P, _P)], y_v)
        pltpu.sync_copy(z_hbm.at[pl.ds(b * _P, _P)], z_v)

        def init_chunk(i, _):
            for u in range(8):
                dist_v[pl.ds(i * (8 * _L) + u * _L, _L)] = jnp.full(
                    (_L,), jnp.inf, jnp.float32)
            return 0

        lax.fori_loop(jnp.int32(0), jnp.int32(_CH // 8), init_chunk, 0)

        plsc.store_scatter(idx_v, [jnp.zeros((_L,), jnp.int32)],
                           jnp.full((_L,), b * _P, jnp.int32),
                           mask=lanes == 0)

        def outer(i, sel):
            selv = jnp.full((_L,), sel, jnp.int32)
            sx = plsc.load_gather(x_v, [selv])
            sy = plsc.load_gather(y_v, [selv])
            sz = plsc.load_gather(z_v, [selv])

            def chunk(k8, carry):
                rmaxs, ridxs = carry
                new_rmaxs, new_ridxs = [], []
                for u in range(8):
                    k = k8 * 8 + u
                    sl = pl.ds(k * _L, _L)
                    dx = x_v[sl] - sx
                    dy = y_v[sl] - sy
                    dz = z_v[sl] - sz
                    d = dx * dx + dy * dy + dz * dz
                    dmin = jnp.minimum(dist_v[sl], d)
                    dist_v[sl] = dmin
                    pred = dmin > rmaxs[u]
                    new_rmaxs.append(jnp.where(pred, dmin, rmaxs[u]))
                    new_ridxs.append(
                        jnp.where(pred, jnp.full((_L,), k, jnp.int32),
                                  ridxs[u]))
                return tuple(new_rmaxs), tuple(new_ridxs)

            neg = jnp.full((_L,), -1.0, jnp.float32)
            zero = jnp.zeros((_L,), jnp.int32)
            rmaxs, ridxs = plsc.parallel_loop(
                jnp.int32(0), jnp.int32(_CH // 8), jnp.int32(1),
                carry=((neg,) * 8, (zero,) * 8))(chunk)

            m8 = rmaxs[0]
            for u in range(1, 8):
                m8 = jnp.maximum(m8, rmaxs[u])
            m = jnp.max(m8)
            big = jnp.full((_L,), 2 ** 30, jnp.int32)
            cand = big
            for u in range(8):
                lin = ridxs[u] * _L + lanes
                cand = jnp.minimum(cand, jnp.where(rmaxs[u] == m, lin, big))
            nsel = jnp.min(cand)
            plsc.store_scatter(idx_v, [jnp.full((_L,), i, jnp.int32)],
                               jnp.full((_L,), b * _P + nsel, jnp.int32),
                               mask=lanes == 0)
            return nsel

        lax.fori_loop(jnp.int32(1), jnp.int32(_S), outer, jnp.int32(0))
        pltpu.sync_copy(idx_v, out_hbm.at[b])


_fps_kernel = functools.partial(
    pl.kernel,
    out_type=jax.ShapeDtypeStruct((_B, _S), jnp.int32),
    mesh=plsc.VectorSubcoreMesh(core_axis_name="c", subcore_axis_name="s",
                                num_cores=2, num_subcores=16),
    compiler_params=pltpu.CompilerParams(needs_layout_passes=False),
    scratch_types=[
        pltpu.VMEM((_P,), jnp.float32),
        pltpu.VMEM((_P,), jnp.float32),
        pltpu.VMEM((_P,), jnp.float32),
        pltpu.VMEM((_P,), jnp.float32),
        pltpu.VMEM((_S,), jnp.int32),
    ],
)(_fps_body)


def kernel(pos, batch):
    del batch
    x = pos[:, 0]
    y = pos[:, 1]
    z = pos[:, 2]
    idx = _fps_kernel(x, y, z)
    return idx.reshape(-1).astype(jnp.int64)
